# Initial kernel scaffold; baseline (speedup 1.0000x reference)
import jax
import jax.numpy as jnp
from jax.experimental import pallas as pl


def kernel(x, edge_index, W1, b1, W2, b2, W3, b3, Wc, bc):
    raise NotImplementedError("write your pallas kernel here")


# speedup vs baseline: 11.0259x; 11.0259x over previous
import functools

import jax
import jax.numpy as jnp
from jax import lax
from jax.experimental import pallas as pl
from jax.experimental.pallas import tpu as pltpu
from jax.experimental.pallas import tpu_sc as plsc

N = 10000
E = 320000
D = 128
NC = 2
NS = 16
NW = NC * NS
EPW = E // NW
BLK = 80
NBLK = EPW // BLK
NP = 10240
RPS = NP // NS
DEGW = 16

_mesh = lambda: plsc.VectorSubcoreMesh(core_axis_name="c", subcore_axis_name="s")


def _sc_degree(dst, ones_blk, zero_rows):
    @functools.partial(
        pl.kernel,
        mesh=_mesh(),
        out_type=jax.ShapeDtypeStruct((NC, NP, DEGW), jnp.float32),
        scratch_types=[
            pltpu.VMEM((BLK,), jnp.int32),
            pltpu.VMEM((BLK, DEGW), jnp.float32),
            pltpu.VMEM_SHARED((NP, DEGW), jnp.float32),
        ],
    )
    def k(dst_hbm, ones_hbm, zeros_hbm, out_hbm, didx, ones_v, acc):
        c = lax.axis_index("c")
        s = lax.axis_index("s")
        pltpu.sync_copy(ones_hbm, ones_v)
        pltpu.sync_copy(zeros_hbm, acc.at[pl.ds(s * RPS, RPS)])
        plsc.subcore_barrier()
        base = (s * NC + c) * EPW

        @pl.loop(0, NBLK)
        def _(i):
            pltpu.sync_copy(dst_hbm.at[pl.ds(base + i * BLK, BLK)], didx)
            pltpu.sync_copy(ones_v, acc.at[didx], add=True)

        plsc.subcore_barrier()
        pltpu.sync_copy(
            acc.at[pl.ds(s * RPS, RPS)], out_hbm.at[c].at[pl.ds(s * RPS, RPS)]
        )

    return k(dst, ones_blk, zero_rows)


def _sc_scatter(hs, src, dst, zero_rows):
    @functools.partial(
        pl.kernel,
        mesh=_mesh(),
        out_type=jax.ShapeDtypeStruct((NC, NP, D), jnp.float32),
        scratch_types=[
            pltpu.VMEM((BLK,), jnp.int32),
            pltpu.VMEM((BLK,), jnp.int32),
            pltpu.VMEM((BLK, D), jnp.float32),
            pltpu.VMEM_SHARED((NP, D), jnp.float32),
            pltpu.SemaphoreType.DMA,
        ],
    )
    def k(hs_hbm, src_hbm, dst_hbm, zeros_hbm, out_hbm, sidx, didx, rows, acc, sem):
        c = lax.axis_index("c")
        s = lax.axis_index("s")
        pltpu.sync_copy(zeros_hbm, acc.at[pl.ds(s * RPS, RPS)])
        plsc.subcore_barrier()
        base = (s * NC + c) * EPW

        @pl.loop(0, NBLK)
        def _(i):
            off = base + i * BLK
            pltpu.sync_copy(src_hbm.at[pl.ds(off, BLK)], sidx)
            pltpu.sync_copy(dst_hbm.at[pl.ds(off, BLK)], didx)
            pltpu.async_copy(hs_hbm.at[sidx], rows, sem).wait()
            pltpu.sync_copy(rows, acc.at[didx], add=True)

        plsc.subcore_barrier()
        pltpu.sync_copy(
            acc.at[pl.ds(s * RPS, RPS)], out_hbm.at[c].at[pl.ds(s * RPS, RPS)]
        )

    return k(hs, src, dst, zero_rows)


_ROWS = 2000


def _dinv_block(degp_ref):
    deg = degp_ref[0, :, 0:1] + degp_ref[1, :, 0:1] + 1.0
    return lax.rsqrt(deg)


def _dot(a, b):
    return lax.dot_general(
        a, b, (((1,), (0,)), ((), ())),
        precision=lax.Precision.HIGHEST,
        preferred_element_type=jnp.float32,
    )


def _tc_first(x, W1, degp):
    def body(x_ref, w_ref, degp_ref, o_ref):
        o_ref[...] = _dinv_block(degp_ref) * _dot(x_ref[...], w_ref[...])

    return pl.pallas_call(
        body,
        grid=(N // _ROWS,),
        in_specs=[
            pl.BlockSpec((_ROWS, D), lambda i: (i, 0)),
            pl.BlockSpec((D, D), lambda i: (0, 0)),
            pl.BlockSpec((NC, _ROWS, DEGW), lambda i: (0, i, 0)),
        ],
        out_specs=pl.BlockSpec((_ROWS, D), lambda i: (i, 0)),
        out_shape=jax.ShapeDtypeStruct((N, D), jnp.float32),
    )(x, W1, degp)


def _tc_mid(accp, hs, degp, b, Wn):
    def body(accp_ref, hs_ref, degp_ref, b_ref, w_ref, o_ref):
        dinv = _dinv_block(degp_ref)
        pre = dinv * (accp_ref[0] + accp_ref[1] + hs_ref[...]) + b_ref[...]
        o_ref[...] = dinv * _dot(jnp.maximum(pre, 0.0), w_ref[...])

    return pl.pallas_call(
        body,
        grid=(N // _ROWS,),
        in_specs=[
            pl.BlockSpec((NC, _ROWS, D), lambda i: (0, i, 0)),
            pl.BlockSpec((_ROWS, D), lambda i: (i, 0)),
            pl.BlockSpec((NC, _ROWS, DEGW), lambda i: (0, i, 0)),
            pl.BlockSpec((1, D), lambda i: (0, 0)),
            pl.BlockSpec((D, D), lambda i: (0, 0)),
        ],
        out_specs=pl.BlockSpec((_ROWS, D), lambda i: (i, 0)),
        out_shape=jax.ShapeDtypeStruct((N, D), jnp.float32),
    )(accp, hs, degp, b, Wn)


def _tc_last(accp, hs, degp, b, Wcp, bcp, n_classes):
    def body(accp_ref, hs_ref, degp_ref, b_ref, w_ref, bc_ref, o_ref):
        dinv = _dinv_block(degp_ref)
        pre = dinv * (accp_ref[0] + accp_ref[1] + hs_ref[...]) + b_ref[...]
        lg = _dot(jnp.maximum(pre, 0.0), w_ref[...])[:, 0:n_classes]
        lg = lg + bc_ref[0, 0:n_classes]
        m = jnp.max(lg, axis=1, keepdims=True)
        lse = m + jnp.log(jnp.sum(jnp.exp(lg - m), axis=1, keepdims=True))
        o_ref[...] = lg - lse

    return pl.pallas_call(
        body,
        grid=(N // _ROWS,),
        in_specs=[
            pl.BlockSpec((NC, _ROWS, D), lambda i: (0, i, 0)),
            pl.BlockSpec((_ROWS, D), lambda i: (i, 0)),
            pl.BlockSpec((NC, _ROWS, DEGW), lambda i: (0, i, 0)),
            pl.BlockSpec((1, D), lambda i: (0, 0)),
            pl.BlockSpec((D, D), lambda i: (0, 0)),
            pl.BlockSpec((1, D), lambda i: (0, 0)),
        ],
        out_specs=pl.BlockSpec((_ROWS, n_classes), lambda i: (i, 0)),
        out_shape=jax.ShapeDtypeStruct((N, n_classes), jnp.float32),
    )(accp, hs, degp, b, Wcp, bcp)


def kernel(x, edge_index, W1, b1, W2, b2, W3, b3, Wc, bc):
    src = edge_index[0]
    dst = edge_index[1]
    n_classes = Wc.shape[1]

    ones_blk = jnp.ones((BLK, DEGW), jnp.float32)
    zero_deg = jnp.zeros((RPS, DEGW), jnp.float32)
    zero_acc = jnp.zeros((RPS, D), jnp.float32)
    b1r = b1.reshape(1, D)
    b2r = b2.reshape(1, D)
    b3r = b3.reshape(1, D)
    Wcp = jnp.zeros((D, D), jnp.float32).at[:, :n_classes].set(Wc)
    bcp = jnp.zeros((1, D), jnp.float32).at[0, :n_classes].set(bc)

    degp = _sc_degree(dst, ones_blk, zero_deg)

    hs1 = _tc_first(x, W1, degp)
    acc1 = _sc_scatter(hs1, src, dst, zero_acc)
    hs2 = _tc_mid(acc1, hs1, degp, b1r, W2)
    acc2 = _sc_scatter(hs2, src, dst, zero_acc)
    hs3 = _tc_mid(acc2, hs2, degp, b2r, W3)
    acc3 = _sc_scatter(hs3, src, dst, zero_acc)
    return _tc_last(acc3, hs3, degp, b3r, Wcp, bcp, n_classes)


# speedup vs baseline: 24.5081x; 2.2228x over previous
import functools

import jax
import jax.numpy as jnp
from jax import lax
from jax.experimental import pallas as pl
from jax.experimental.pallas import tpu as pltpu
from jax.experimental.pallas import tpu_sc as plsc

N = 10000
E = 320000
D = 128
NC = 2
NS = 16
NW = NC * NS
EPW = E // NW
BLK = 80
NBLK = EPW // BLK
NP = 10240
RPS = NP // NS

_mesh = lambda: plsc.VectorSubcoreMesh(core_axis_name="c", subcore_axis_name="s")


NIDX = 8
NSLOT = 4
DEGW = 128


def _sc_degree(dst, ones_blk, zero_rows):
    @functools.partial(
        pl.kernel,
        mesh=_mesh(),
        out_type=jax.ShapeDtypeStruct((NC, NP, DEGW), jnp.float32),
        scratch_types=[
            [pltpu.VMEM((BLK,), jnp.int32)] * 1,
            pltpu.VMEM((BLK, DEGW), jnp.float32),
            pltpu.VMEM_SHARED((NP, DEGW), jnp.float32),
        ],
    )
    def k(dst_hbm, ones_hbm, zeros_hbm, out_hbm, didx, ones_v, acc):
        c = lax.axis_index("c")
        s = lax.axis_index("s")
        base = (s * NC + c) * EPW
        pltpu.sync_copy(ones_hbm, ones_v)
        pltpu.sync_copy(zeros_hbm, acc.at[pl.ds(s * RPS, RPS)])
        plsc.subcore_barrier()

        @pl.loop(0, NBLK)
        def _(i):
            pltpu.sync_copy(dst_hbm.at[pl.ds(base + i * BLK, BLK)], didx[0])
            pltpu.sync_copy(ones_v, acc.at[didx[0]], add=True)

        plsc.subcore_barrier()
        pltpu.sync_copy(
            acc.at[pl.ds(s * RPS, RPS)], out_hbm.at[c].at[pl.ds(s * RPS, RPS)]
        )

    return k(dst, ones_blk, zero_rows)


def _sc_scatter(hs, src, dst, zero_rows):
    @functools.partial(
        pl.kernel,
        mesh=_mesh(),
        out_type=jax.ShapeDtypeStruct((NC, NP, D), jnp.float32),
        scratch_types=[
            [pltpu.VMEM((BLK,), jnp.int32)] * NIDX,
            [pltpu.VMEM((BLK,), jnp.int32)] * NIDX,
            pltpu.VMEM((NSLOT, BLK, D), jnp.float32),
            pltpu.VMEM_SHARED((NP, D), jnp.float32),
            [pltpu.SemaphoreType.DMA] * NIDX,
            [pltpu.SemaphoreType.DMA] * NSLOT,
        ],
    )
    def k(hs_hbm, src_hbm, dst_hbm, zeros_hbm, out_hbm, sidx, didx, rows, acc,
          isems, gsems):
        c = lax.axis_index("c")
        s = lax.axis_index("s")
        base = (s * NC + c) * EPW
        pltpu.sync_copy(zeros_hbm, acc.at[pl.ds(s * RPS, RPS)])
        plsc.subcore_barrier()

        def start_idx(q, i):
            pltpu.async_copy(src_hbm.at[pl.ds(base + i * BLK, BLK)], sidx[q],
                             isems[q])
            pltpu.async_copy(dst_hbm.at[pl.ds(base + i * BLK, BLK)], didx[q],
                             isems[q])

        def wait_idx(q):
            pltpu.make_async_copy(src_hbm.at[pl.ds(base, BLK)], sidx[q],
                                  isems[q]).wait()
            pltpu.make_async_copy(dst_hbm.at[pl.ds(base, BLK)], didx[q],
                                  isems[q]).wait()

        def start_gather(b, q):
            pltpu.async_copy(hs_hbm.at[sidx[q]], rows.at[b], gsems[b])

        def wait_gather(b, q):
            pltpu.make_async_copy(hs_hbm.at[sidx[q]], rows.at[b],
                                  gsems[b]).wait()

        for q in range(NIDX):
            start_idx(q, q)
        for b in range(NSLOT):
            wait_idx(b)
            start_gather(b, b)

        M = (NBLK - 2 * NIDX) // NIDX

        @pl.loop(0, M)
        def _(kk):
            i0 = kk * NIDX
            for u in range(NIDX):
                b, q = u % NSLOT, u
                wait_gather(b, q)
                pltpu.sync_copy(rows.at[b], acc.at[didx[q]], add=True)
                start_idx(q, i0 + NIDX + u)
                q2 = (u + NSLOT) % NIDX
                wait_idx(q2)
                start_gather(b, q2)

        for i in range(M * NIDX, NBLK):
            b, q = i % NSLOT, i % NIDX
            wait_gather(b, q)
            pltpu.sync_copy(rows.at[b], acc.at[didx[q]], add=True)
            if i + NIDX < NBLK:
                start_idx(q, i + NIDX)
            if i + NSLOT < NBLK:
                q2 = (i + NSLOT) % NIDX
                wait_idx(q2)
                start_gather(b, q2)

        plsc.subcore_barrier()
        pltpu.sync_copy(
            acc.at[pl.ds(s * RPS, RPS)], out_hbm.at[c].at[pl.ds(s * RPS, RPS)]
        )

    return k(hs, src, dst, zero_rows)


_ROWS = 2000


def _dinv_block(degp_ref):
    deg = degp_ref[0, :, 0:1] + degp_ref[1, :, 0:1] + 1.0
    return lax.rsqrt(deg)


def _dot(a, b):
    return lax.dot_general(
        a, b, (((1,), (0,)), ((), ())),
        precision=lax.Precision.HIGHEST,
        preferred_element_type=jnp.float32,
    )


def _tc_first(x, W1, degp):
    def body(x_ref, w_ref, degp_ref, o_ref):
        o_ref[...] = _dinv_block(degp_ref) * _dot(x_ref[...], w_ref[...])

    return pl.pallas_call(
        body,
        grid=(N // _ROWS,),
        in_specs=[
            pl.BlockSpec((_ROWS, D), lambda i: (i, 0)),
            pl.BlockSpec((D, D), lambda i: (0, 0)),
            pl.BlockSpec((NC, _ROWS, DEGW), lambda i: (0, i, 0)),
        ],
        out_specs=pl.BlockSpec((_ROWS, D), lambda i: (i, 0)),
        out_shape=jax.ShapeDtypeStruct((N, D), jnp.float32),
    )(x, W1, degp)


def _tc_mid(accp, hs, degp, b, Wn):
    def body(accp_ref, hs_ref, degp_ref, b_ref, w_ref, o_ref):
        dinv = _dinv_block(degp_ref)
        pre = dinv * (accp_ref[0] + accp_ref[1] + hs_ref[...]) + b_ref[...]
        o_ref[...] = dinv * _dot(jnp.maximum(pre, 0.0), w_ref[...])

    return pl.pallas_call(
        body,
        grid=(N // _ROWS,),
        in_specs=[
            pl.BlockSpec((NC, _ROWS, D), lambda i: (0, i, 0)),
            pl.BlockSpec((_ROWS, D), lambda i: (i, 0)),
            pl.BlockSpec((NC, _ROWS, DEGW), lambda i: (0, i, 0)),
            pl.BlockSpec((1, D), lambda i: (0, 0)),
            pl.BlockSpec((D, D), lambda i: (0, 0)),
        ],
        out_specs=pl.BlockSpec((_ROWS, D), lambda i: (i, 0)),
        out_shape=jax.ShapeDtypeStruct((N, D), jnp.float32),
    )(accp, hs, degp, b, Wn)


def _tc_last(accp, hs, degp, b, Wcp, bcp, n_classes):
    def body(accp_ref, hs_ref, degp_ref, b_ref, w_ref, bc_ref, o_ref):
        dinv = _dinv_block(degp_ref)
        pre = dinv * (accp_ref[0] + accp_ref[1] + hs_ref[...]) + b_ref[...]
        lg = _dot(jnp.maximum(pre, 0.0), w_ref[...])[:, 0:n_classes]
        lg = lg + bc_ref[0, 0:n_classes]
        m = jnp.max(lg, axis=1, keepdims=True)
        lse = m + jnp.log(jnp.sum(jnp.exp(lg - m), axis=1, keepdims=True))
        o_ref[...] = lg - lse

    return pl.pallas_call(
        body,
        grid=(N // _ROWS,),
        in_specs=[
            pl.BlockSpec((NC, _ROWS, D), lambda i: (0, i, 0)),
            pl.BlockSpec((_ROWS, D), lambda i: (i, 0)),
            pl.BlockSpec((NC, _ROWS, DEGW), lambda i: (0, i, 0)),
            pl.BlockSpec((1, D), lambda i: (0, 0)),
            pl.BlockSpec((D, D), lambda i: (0, 0)),
            pl.BlockSpec((1, D), lambda i: (0, 0)),
        ],
        out_specs=pl.BlockSpec((_ROWS, n_classes), lambda i: (i, 0)),
        out_shape=jax.ShapeDtypeStruct((N, n_classes), jnp.float32),
    )(accp, hs, degp, b, Wcp, bcp)


def kernel(x, edge_index, W1, b1, W2, b2, W3, b3, Wc, bc):
    src = edge_index[0]
    dst = edge_index[1]
    n_classes = Wc.shape[1]

    ones_blk = jnp.ones((BLK, DEGW), jnp.float32)
    zero_deg = jnp.zeros((RPS, DEGW), jnp.float32)
    zero_acc = jnp.zeros((RPS, D), jnp.float32)
    b1r = b1.reshape(1, D)
    b2r = b2.reshape(1, D)
    b3r = b3.reshape(1, D)
    Wcp = jnp.zeros((D, D), jnp.float32).at[:, :n_classes].set(Wc)
    bcp = jnp.zeros((1, D), jnp.float32).at[0, :n_classes].set(bc)

    degp = _sc_degree(dst, ones_blk, zero_deg)

    hs1 = _tc_first(x, W1, degp)
    acc1 = _sc_scatter(hs1, src, dst, zero_acc)
    hs2 = _tc_mid(acc1, hs1, degp, b1r, W2)
    acc2 = _sc_scatter(hs2, src, dst, zero_acc)
    hs3 = _tc_mid(acc2, hs2, degp, b2r, W3)
    acc3 = _sc_scatter(hs3, src, dst, zero_acc)
    return _tc_last(acc3, hs3, degp, b3r, Wcp, bcp, n_classes)


# speedup vs baseline: 25.8096x; 1.0531x over previous
import functools

import jax
import jax.numpy as jnp
from jax import lax
from jax.experimental import pallas as pl
from jax.experimental.pallas import tpu as pltpu
from jax.experimental.pallas import tpu_sc as plsc

N = 10000
E = 320000
D = 128
NC = 2
NS = 16
NW = NC * NS
EPW = E // NW
BLK = 80
NBLK = EPW // BLK
NP = 10240
RPS = NP // NS

_mesh = lambda: plsc.VectorSubcoreMesh(core_axis_name="c", subcore_axis_name="s")


NIDX = 8
NSLOT = 4
DEGW = 128


def _sc_degree(dst, ones_blk, zero_rows):
    @functools.partial(
        pl.kernel,
        mesh=_mesh(),
        out_type=jax.ShapeDtypeStruct((NC, NP, DEGW), jnp.float32),
        scratch_types=[
            [pltpu.VMEM((BLK,), jnp.int32)] * 2,
            pltpu.VMEM((BLK, DEGW), jnp.float32),
            pltpu.VMEM_SHARED((NP, DEGW), jnp.float32),
            [pltpu.SemaphoreType.DMA] * 2,
        ],
    )
    def k(dst_hbm, ones_hbm, zeros_hbm, out_hbm, didx, ones_v, acc, isems):
        c = lax.axis_index("c")
        s = lax.axis_index("s")
        base = (s * NC + c) * EPW
        pltpu.sync_copy(ones_hbm, ones_v)
        pltpu.sync_copy(zeros_hbm, acc.at[pl.ds(s * RPS, RPS)])
        plsc.subcore_barrier()

        @pl.loop(0, NBLK // 2)
        def _(kk):
            i0 = 2 * kk
            d0 = pltpu.async_copy(
                dst_hbm.at[pl.ds(base + i0 * BLK, BLK)], didx[0], isems[0])
            d1 = pltpu.async_copy(
                dst_hbm.at[pl.ds(base + (i0 + 1) * BLK, BLK)], didx[1], isems[1])
            d0.wait()
            pltpu.sync_copy(ones_v, acc.at[didx[0]], add=True)
            d1.wait()
            pltpu.sync_copy(ones_v, acc.at[didx[1]], add=True)

        for i in range((NBLK // 2) * 2, NBLK):
            pltpu.sync_copy(dst_hbm.at[pl.ds(base + i * BLK, BLK)], didx[0])
            pltpu.sync_copy(ones_v, acc.at[didx[0]], add=True)

        plsc.subcore_barrier()
        pltpu.sync_copy(
            acc.at[pl.ds(s * RPS, RPS)], out_hbm.at[c].at[pl.ds(s * RPS, RPS)]
        )

    return k(dst, ones_blk, zero_rows)


def _sc_scatter(hs, src, dst, zero_rows):
    @functools.partial(
        pl.kernel,
        mesh=_mesh(),
        out_type=jax.ShapeDtypeStruct((NC, NP, D), jnp.float32),
        scratch_types=[
            [pltpu.VMEM((BLK,), jnp.int32)] * NIDX,
            [pltpu.VMEM((BLK,), jnp.int32)] * NIDX,
            pltpu.VMEM((NSLOT, BLK, D), jnp.float32),
            pltpu.VMEM_SHARED((NP, D), jnp.float32),
            [pltpu.SemaphoreType.DMA] * NIDX,
            [pltpu.SemaphoreType.DMA] * NSLOT,
        ],
    )
    def k(hs_hbm, src_hbm, dst_hbm, zeros_hbm, out_hbm, sidx, didx, rows, acc,
          isems, gsems):
        c = lax.axis_index("c")
        s = lax.axis_index("s")
        base = (s * NC + c) * EPW
        pltpu.sync_copy(zeros_hbm, acc.at[pl.ds(s * RPS, RPS)])
        plsc.subcore_barrier()

        def start_idx(q, i):
            pltpu.async_copy(src_hbm.at[pl.ds(base + i * BLK, BLK)], sidx[q],
                             isems[q])
            pltpu.async_copy(dst_hbm.at[pl.ds(base + i * BLK, BLK)], didx[q],
                             isems[q])

        def wait_idx(q):
            pltpu.make_async_copy(src_hbm.at[pl.ds(base, BLK)], sidx[q],
                                  isems[q]).wait()
            pltpu.make_async_copy(dst_hbm.at[pl.ds(base, BLK)], didx[q],
                                  isems[q]).wait()

        def start_gather(b, q):
            pltpu.async_copy(hs_hbm.at[sidx[q]], rows.at[b], gsems[b])

        def wait_gather(b, q):
            pltpu.make_async_copy(hs_hbm.at[sidx[q]], rows.at[b],
                                  gsems[b]).wait()

        for q in range(NIDX):
            start_idx(q, q)
        for b in range(NSLOT):
            wait_idx(b)
            start_gather(b, b)

        M = (NBLK - 2 * NIDX) // NIDX

        @pl.loop(0, M)
        def _(kk):
            i0 = kk * NIDX
            for u in range(NIDX):
                b, q = u % NSLOT, u
                wait_gather(b, q)
                pltpu.sync_copy(rows.at[b], acc.at[didx[q]], add=True)
                start_idx(q, i0 + NIDX + u)
                q2 = (u + NSLOT) % NIDX
                wait_idx(q2)
                start_gather(b, q2)

        for i in range(M * NIDX, NBLK):
            b, q = i % NSLOT, i % NIDX
            wait_gather(b, q)
            pltpu.sync_copy(rows.at[b], acc.at[didx[q]], add=True)
            if i + NIDX < NBLK:
                start_idx(q, i + NIDX)
            if i + NSLOT < NBLK:
                q2 = (i + NSLOT) % NIDX
                wait_idx(q2)
                start_gather(b, q2)

        plsc.subcore_barrier()
        pltpu.sync_copy(
            acc.at[pl.ds(s * RPS, RPS)], out_hbm.at[c].at[pl.ds(s * RPS, RPS)]
        )

    return k(hs, src, dst, zero_rows)


_ROWS = 2000


def _dinv_block(degp_ref):
    deg = degp_ref[0, :, 0:1] + degp_ref[1, :, 0:1] + 1.0
    return lax.rsqrt(deg)


def _dot(a, b):
    return lax.dot_general(
        a, b, (((1,), (0,)), ((), ())),
        precision=lax.Precision.HIGHEST,
        preferred_element_type=jnp.float32,
    )


def _tc_first(x, W1, degp):
    def body(x_ref, w_ref, degp_ref, o_ref):
        o_ref[...] = _dinv_block(degp_ref) * _dot(x_ref[...], w_ref[...])

    return pl.pallas_call(
        body,
        grid=(N // _ROWS,),
        in_specs=[
            pl.BlockSpec((_ROWS, D), lambda i: (i, 0)),
            pl.BlockSpec((D, D), lambda i: (0, 0)),
            pl.BlockSpec((NC, _ROWS, DEGW), lambda i: (0, i, 0)),
        ],
        out_specs=pl.BlockSpec((_ROWS, D), lambda i: (i, 0)),
        out_shape=jax.ShapeDtypeStruct((N, D), jnp.float32),
    )(x, W1, degp)


def _tc_mid(accp, hs, degp, b, Wn):
    def body(accp_ref, hs_ref, degp_ref, b_ref, w_ref, o_ref):
        dinv = _dinv_block(degp_ref)
        pre = dinv * (accp_ref[0] + accp_ref[1] + hs_ref[...]) + b_ref[...]
        o_ref[...] = dinv * _dot(jnp.maximum(pre, 0.0), w_ref[...])

    return pl.pallas_call(
        body,
        grid=(N // _ROWS,),
        in_specs=[
            pl.BlockSpec((NC, _ROWS, D), lambda i: (0, i, 0)),
            pl.BlockSpec((_ROWS, D), lambda i: (i, 0)),
            pl.BlockSpec((NC, _ROWS, DEGW), lambda i: (0, i, 0)),
            pl.BlockSpec((1, D), lambda i: (0, 0)),
            pl.BlockSpec((D, D), lambda i: (0, 0)),
        ],
        out_specs=pl.BlockSpec((_ROWS, D), lambda i: (i, 0)),
        out_shape=jax.ShapeDtypeStruct((N, D), jnp.float32),
    )(accp, hs, degp, b, Wn)


def _tc_last(accp, hs, degp, b, Wcp, bcp, n_classes):
    def body(accp_ref, hs_ref, degp_ref, b_ref, w_ref, bc_ref, o_ref):
        dinv = _dinv_block(degp_ref)
        pre = dinv * (accp_ref[0] + accp_ref[1] + hs_ref[...]) + b_ref[...]
        lg = _dot(jnp.maximum(pre, 0.0), w_ref[...])[:, 0:n_classes]
        lg = lg + bc_ref[0, 0:n_classes]
        m = jnp.max(lg, axis=1, keepdims=True)
        lse = m + jnp.log(jnp.sum(jnp.exp(lg - m), axis=1, keepdims=True))
        o_ref[...] = lg - lse

    return pl.pallas_call(
        body,
        grid=(N // _ROWS,),
        in_specs=[
            pl.BlockSpec((NC, _ROWS, D), lambda i: (0, i, 0)),
            pl.BlockSpec((_ROWS, D), lambda i: (i, 0)),
            pl.BlockSpec((NC, _ROWS, DEGW), lambda i: (0, i, 0)),
            pl.BlockSpec((1, D), lambda i: (0, 0)),
            pl.BlockSpec((D, D), lambda i: (0, 0)),
            pl.BlockSpec((1, D), lambda i: (0, 0)),
        ],
        out_specs=pl.BlockSpec((_ROWS, n_classes), lambda i: (i, 0)),
        out_shape=jax.ShapeDtypeStruct((N, n_classes), jnp.float32),
    )(accp, hs, degp, b, Wcp, bcp)


def kernel(x, edge_index, W1, b1, W2, b2, W3, b3, Wc, bc):
    src = edge_index[0]
    dst = edge_index[1]
    n_classes = Wc.shape[1]

    ones_blk = jnp.ones((BLK, DEGW), jnp.float32)
    zero_deg = jnp.zeros((RPS, DEGW), jnp.float32)
    zero_acc = jnp.zeros((RPS, D), jnp.float32)
    b1r = b1.reshape(1, D)
    b2r = b2.reshape(1, D)
    b3r = b3.reshape(1, D)
    Wcp = jnp.zeros((D, D), jnp.float32).at[:, :n_classes].set(Wc)
    bcp = jnp.zeros((1, D), jnp.float32).at[0, :n_classes].set(bc)

    degp = _sc_degree(dst, ones_blk, zero_deg)

    hs1 = _tc_first(x, W1, degp)
    acc1 = _sc_scatter(hs1, src, dst, zero_acc)
    hs2 = _tc_mid(acc1, hs1, degp, b1r, W2)
    acc2 = _sc_scatter(hs2, src, dst, zero_acc)
    hs3 = _tc_mid(acc2, hs2, degp, b2r, W3)
    acc3 = _sc_scatter(hs3, src, dst, zero_acc)
    return _tc_last(acc3, hs3, degp, b3r, Wcp, bcp, n_classes)
